# Initial kernel scaffold; baseline (speedup 1.0000x reference)
import functools

import jax
import jax.numpy as jnp
from jax import lax
from jax.experimental import pallas as pl
from jax.experimental.pallas import tpu as pltpu
from jax.experimental.pallas import tpu_sc as plsc

_NUM_CORES = 2
_NUM_SUBCORES = 16
_NUM_WORKERS = _NUM_CORES * _NUM_SUBCORES

_CHUNK = 512


@functools.partial(jax.jit, static_argnames=("b", "d"))
def _gather(flat_ids, table, *, b, d):
    b_per_w = b // _NUM_WORKERS
    n_chunks = b_per_w // _CHUNK
    mesh = plsc.VectorSubcoreMesh(core_axis_name="c", subcore_axis_name="s")

    @functools.partial(
        pl.kernel,
        mesh=mesh,
        out_type=jax.ShapeDtypeStruct((b, d), jnp.float32),
        scratch_types=[
            pltpu.VMEM((_CHUNK,), jnp.int32),
            pltpu.VMEM((_CHUNK, d), jnp.float32),
            pltpu.SemaphoreType.DMA,
        ],
    )
    def k(idx_hbm, table_hbm, out_hbm, idx_v, rows_v, sem):
        wid = lax.axis_index("s") * _NUM_CORES + lax.axis_index("c")
        base = wid * b_per_w

        def body(c, carry):
            off = base + c * _CHUNK
            pltpu.sync_copy(idx_hbm.at[pl.ds(off, _CHUNK)], idx_v)
            pltpu.async_copy(table_hbm.at[idx_v], rows_v, sem).wait()
            pltpu.sync_copy(rows_v, out_hbm.at[pl.ds(off, _CHUNK)])
            return carry

        lax.fori_loop(0, n_chunks, body, 0)

    return k(flat_ids, table)


def kernel(token_ids, embedding_table):
    batch, hist = token_ids.shape
    vocab, d = embedding_table.shape
    flat_ids = token_ids.reshape(batch * hist).astype(jnp.int32)
    out = _gather(flat_ids, embedding_table, b=batch * hist, d=d)
    return out.reshape(batch, hist, d)


# speedup vs baseline: 1.8670x; 1.8670x over previous
import functools

import jax
import jax.numpy as jnp
from jax import lax
from jax.experimental import pallas as pl
from jax.experimental.pallas import tpu as pltpu
from jax.experimental.pallas import tpu_sc as plsc

_NUM_CORES = 2
_NUM_SUBCORES = 16
_NUM_WORKERS = _NUM_CORES * _NUM_SUBCORES

_CHUNK = 320
_NBUF = 4


@functools.partial(jax.jit, static_argnames=("b", "d"))
def _gather(flat_ids, table, *, b, d):
    b_per_w = b // _NUM_WORKERS
    n_chunks = b_per_w // _CHUNK
    n_rounds = n_chunks // _NBUF
    mesh = plsc.VectorSubcoreMesh(core_axis_name="c", subcore_axis_name="s")

    @functools.partial(
        pl.kernel,
        mesh=mesh,
        out_type=jax.ShapeDtypeStruct((b, d), jnp.float32),
        scratch_types=[
            pltpu.VMEM((b_per_w,), jnp.int32),
            [pltpu.VMEM((_CHUNK, d), jnp.float32) for _ in range(_NBUF)],
            [pltpu.SemaphoreType.DMA for _ in range(_NBUF)],
            [pltpu.SemaphoreType.DMA for _ in range(_NBUF)],
            pltpu.SemaphoreType.DMA,
        ],
        compiler_params=pltpu.CompilerParams(use_tc_tiling_on_sc=False),
    )
    def k(idx_hbm, table_hbm, out_hbm, idx_v, rows, sem_g, sem_s, sem_i):
        wid = lax.axis_index("s") * _NUM_CORES + lax.axis_index("c")
        base = wid * b_per_w
        pltpu.async_copy(idx_hbm.at[pl.ds(base, b_per_w)], idx_v, sem_i).wait()

        def fire_gather(c, bslot):
            pltpu.async_copy(
                table_hbm.at[idx_v.at[pl.ds(c * _CHUNK, _CHUNK)]],
                rows[bslot],
                sem_g[bslot],
            )

        def wait_gather(c, bslot):
            pltpu.make_async_copy(
                table_hbm.at[idx_v.at[pl.ds(c * _CHUNK, _CHUNK)]],
                rows[bslot],
                sem_g[bslot],
            ).wait()

        def fire_store(c, bslot):
            pltpu.async_copy(
                rows[bslot],
                out_hbm.at[pl.ds(base + c * _CHUNK, _CHUNK)],
                sem_s[bslot],
            )

        def wait_store(c, bslot):
            pltpu.make_async_copy(
                rows[bslot],
                out_hbm.at[pl.ds(base + c * _CHUNK, _CHUNK)],
                sem_s[bslot],
            ).wait()

        for bslot in range(_NBUF):
            fire_gather(bslot, bslot)

        def body(j, carry):
            c0 = j * _NBUF
            for bslot in range(_NBUF):
                wait_gather(c0 + bslot, bslot)
                fire_store(c0 + bslot, bslot)
            for bslot in range(_NBUF):
                wait_store(c0 + bslot, bslot)
                fire_gather(c0 + _NBUF + bslot, bslot)
            return carry

        lax.fori_loop(0, n_rounds - 1, body, 0)

        c0 = (n_rounds - 1) * _NBUF
        for bslot in range(_NBUF):
            wait_gather(c0 + bslot, bslot)
            fire_store(c0 + bslot, bslot)
        for bslot in range(_NBUF):
            wait_store(c0 + bslot, bslot)

    return k(flat_ids, table)


def kernel(token_ids, embedding_table):
    batch, hist = token_ids.shape
    vocab, d = embedding_table.shape
    flat_ids = token_ids.reshape(batch * hist).astype(jnp.int32)
    out = _gather(flat_ids, embedding_table, b=batch * hist, d=d)
    return out.reshape(batch, hist, d)
